# baseline (speedup 1.0000x reference)
import jax
import jax.numpy as jnp
from jax.experimental import pallas as pl

_N = 10000
_BATCH = 1


def _isin_sorted(a, b_sorted):
    idx = jnp.clip(jnp.searchsorted(b_sorted, a), 0, b_sorted.shape[0] - 1)
    return b_sorted[idx] == a


def _batch_keys(mask_idx, n0, n, batch_size):
    offs = jnp.arange(batch_size, dtype=mask_idx.dtype) * n0
    r = (mask_idx[0][None, :] + offs[:, None]).reshape(-1)
    c = (mask_idx[1][None, :] + offs[:, None]).reshape(-1)
    return r.astype(jnp.int64) * n + c.astype(jnp.int64)


def _div_kernel(a_ref, d_ref, o_ref):
    o_ref[...] = a_ref[...] / d_ref[...]


def kernel(attr_idx, edge_idx, S_mask1_idx, S_mask2_idx, W1, W2, n, d, n0):
    batch_size = _BATCH
    edge_keys = edge_idx[0].astype(jnp.int64) * n + edge_idx[1].astype(jnp.int64)
    m1 = jnp.sort(_batch_keys(S_mask1_idx, n0, n, batch_size))
    keep = _isin_sorted(edge_keys, m1)
    m2 = _batch_keys(S_mask2_idx, n0, n, batch_size)
    keys = jnp.concatenate([m2, edge_keys])
    valid = jnp.concatenate([jnp.ones(m2.shape[0], dtype=bool), keep])
    sentinel = jnp.asarray(n, keys.dtype) * n
    keys = jnp.where(valid, keys, sentinel)
    keys = jnp.sort(keys)
    first = jnp.concatenate([jnp.ones((1,), dtype=bool), keys[1:] != keys[:-1]])
    ok = keys < sentinel
    w = (first & ok).astype(jnp.float32)
    row = jnp.where(ok, keys // n, 0).astype(jnp.int32)
    col = jnp.where(ok, keys % n, 0).astype(jnp.int32)

    x = jnp.zeros((_N, W1.shape[0]), dtype=jnp.float32).at[attr_idx[0], attr_idx[1] % d].set(1.0)
    deg = jnp.zeros((_N,), jnp.float32).at[row].add(w) + 1.0
    h = x @ W1
    agg = jnp.zeros((_N, W1.shape[1]), jnp.float32).at[row].add(h[col] * w[:, None])
    h1 = jax.nn.relu(agg / deg[:, None])
    h2 = h1 @ W2
    agg2 = jnp.zeros((_N, W2.shape[1]), jnp.float32).at[row].add(h2[col] * w[:, None])

    return pl.pallas_call(
        _div_kernel,
        out_shape=jax.ShapeDtypeStruct((_N, W2.shape[1]), jnp.float32),
    )(agg2, jnp.broadcast_to(deg[:, None], (_N, W2.shape[1])))


# speedup vs baseline: 1.2327x; 1.2327x over previous
import functools

import jax
import jax.numpy as jnp
from jax.experimental import pallas as pl

_NP = 10240
_RT = 1024
_KC = 2048


def _h_kernel(x8_ref, w1_ref, o_ref):
    x = jnp.minimum(x8_ref[...].astype(jnp.float32), 1.0)
    o_ref[...] = jax.lax.dot_general(
        x, w1_ref[...], (((1,), (0,)), ((), ())),
        preferred_element_type=jnp.float32)


def _p1_kernel(u_ref, h_ref, w2_ref, h2_ref, deg_ref, acc_ref, dacc_ref, *, nk):
    k = pl.program_id(1)

    @pl.when(k == 0)
    def _():
        acc_ref[...] = jnp.zeros_like(acc_ref)
        dacc_ref[...] = jnp.zeros_like(dacc_ref)

    m = jnp.minimum(u_ref[...].astype(jnp.float32), 1.0)
    acc_ref[...] += jax.lax.dot_general(
        m, h_ref[...], (((1,), (0,)), ((), ())),
        preferred_element_type=jnp.float32)
    dacc_ref[...] += jnp.sum(m, axis=1, keepdims=True)

    @pl.when(k == nk - 1)
    def _():
        deg = dacc_ref[...] + 1.0
        h1 = jax.nn.relu(acc_ref[...] / deg)
        h2_ref[...] = jax.lax.dot_general(
            h1, w2_ref[...], (((1,), (0,)), ((), ())),
            preferred_element_type=jnp.float32)
        deg_ref[...] = deg


def _p2_kernel(u_ref, h2_ref, deg_ref, o_ref, acc_ref, *, nk):
    k = pl.program_id(1)

    @pl.when(k == 0)
    def _():
        acc_ref[...] = jnp.zeros_like(acc_ref)

    m = jnp.minimum(u_ref[...].astype(jnp.float32), 1.0)
    acc_ref[...] += jax.lax.dot_general(
        m, h2_ref[...], (((1,), (0,)), ((), ())),
        preferred_element_type=jnp.float32)

    @pl.when(k == nk - 1)
    def _():
        o_ref[...] = acc_ref[...] / deg_ref[...]


def kernel(attr_idx, edge_idx, S_mask1_idx, S_mask2_idx, W1, W2, n, d, n0):
    ds = W1.shape[0]
    ns = 10000
    f1 = W1.shape[1]
    f2 = W2.shape[1]
    one = jnp.uint8(1)

    x8 = jnp.zeros((_NP, ds), jnp.uint8).at[attr_idx[0], attr_idx[1] % ds].set(one)
    a1 = jnp.zeros((_NP, _NP), jnp.uint8).at[S_mask1_idx[0], S_mask1_idx[1]].set(one)
    keep = a1[edge_idx[0], edge_idx[1]]
    u = (jnp.zeros((_NP, _NP), jnp.uint8)
         .at[S_mask2_idx[0], S_mask2_idx[1]].set(one)
         .at[edge_idx[0], edge_idx[1]].max(keep))

    ni = _NP // _RT
    nk = _NP // _KC

    h = pl.pallas_call(
        _h_kernel,
        grid=(ni,),
        in_specs=[
            pl.BlockSpec((_RT, ds), lambda i: (i, 0)),
            pl.BlockSpec((ds, f1), lambda i: (0, 0)),
        ],
        out_specs=pl.BlockSpec((_RT, f1), lambda i: (i, 0)),
        out_shape=jax.ShapeDtypeStruct((_NP, f1), jnp.float32),
    )(x8, W1)

    h2, deg = pl.pallas_call(
        functools.partial(_p1_kernel, nk=nk),
        grid=(ni, nk),
        in_specs=[
            pl.BlockSpec((_RT, _KC), lambda i, k: (i, k)),
            pl.BlockSpec((_KC, f1), lambda i, k: (k, 0)),
            pl.BlockSpec((f1, f2), lambda i, k: (0, 0)),
        ],
        out_specs=[
            pl.BlockSpec((_RT, f2), lambda i, k: (i, 0)),
            pl.BlockSpec((_RT, 1), lambda i, k: (i, 0)),
        ],
        out_shape=[
            jax.ShapeDtypeStruct((_NP, f2), jnp.float32),
            jax.ShapeDtypeStruct((_NP, 1), jnp.float32),
        ],
        scratch_shapes=[
            pltpu_vmem((_RT, f1), jnp.float32),
            pltpu_vmem((_RT, 1), jnp.float32),
        ],
    )(u, h, W2)

    out = pl.pallas_call(
        functools.partial(_p2_kernel, nk=nk),
        grid=(ni, nk),
        in_specs=[
            pl.BlockSpec((_RT, _KC), lambda i, k: (i, k)),
            pl.BlockSpec((_KC, f2), lambda i, k: (k, 0)),
            pl.BlockSpec((_RT, 1), lambda i, k: (i, 0)),
        ],
        out_specs=pl.BlockSpec((_RT, f2), lambda i, k: (i, 0)),
        out_shape=jax.ShapeDtypeStruct((_NP, f2), jnp.float32),
        scratch_shapes=[pltpu_vmem((_RT, f2), jnp.float32)],
    )(u, h2, deg)

    return out[:ns]


def pltpu_vmem(shape, dtype):
    from jax.experimental.pallas import tpu as pltpu
    return pltpu.VMEM(shape, dtype)


# speedup vs baseline: 2.8141x; 2.2829x over previous
import functools

import jax
import jax.numpy as jnp
from jax import lax
from jax.experimental import pallas as pl
from jax.experimental.pallas import tpu as pltpu
from jax.experimental.pallas import tpu_sc as plsc

_NP = 10240
_RT = 1024
_KC = 1024
_PLANE = _NP * _NP
_SB = 19
_STRIP = 1 << _SB
_NS_P = _PLANE >> _SB
_XPAD = 5 << _SB
_M2UNIT = float(1 << 19)
_KMASK = (1 << 27) - 1
_CHUNK = 2048
_NVEC = _CHUNK // 16
_RB = 1
_ZB = 16384


def _pad_keys(k, shard_chunks):
    total = shard_chunks * _CHUNK * 16
    pad = total - k.shape[0]
    padk = ((jnp.arange(pad, dtype=jnp.int32) & 8191) * 64) | (1 << 28)
    return jnp.concatenate([k, padk])


def _scal(ref, idx):
    base = pl.multiple_of((idx // 16) * 16, 16)
    vec = ref[pl.ds(base, 16)]
    lane = lax.iota(jnp.int32, 16) == (idx % 16)
    return jnp.max(jnp.where(lane, vec, jnp.int32(0)))


def _build_plane(keys_ref, plane_ref, sub, refs, *, shard_chunks, nstrips):
    (strip_sp, bucket_sp, counts_sp, kbuf, valbuf, zbuf, hist_v, cnt_v,
     base_v, next_v, start_v) = refs
    tl = lax.iota(jnp.int32, 16)
    i32 = jnp.int32
    ngroup = (nstrips + 15) // 16

    def _zero16(g, _):
        cnt_v[pl.ds(g * 16, 16)] = jnp.zeros((16,), i32)
        return 0

    lax.fori_loop(0, 16, _zero16, 0)
    shard0 = sub * shard_chunks * _CHUNK

    def _a_chunk(ch, _):
        off = pl.multiple_of(shard0 + ch * _CHUNK, _CHUNK)
        pltpu.sync_copy(keys_ref.at[pl.ds(off, _CHUNK)], kbuf)

        def _a_vec(i, _):
            kv = kbuf[pl.ds(i * 16, 16)]
            st = (kv & _KMASK) >> _SB
            rank, last = plsc.scan_count(st)
            plsc.addupdate_scatter(cnt_v, [st], rank + (1 - _RB), mask=last)
            return 0

        lax.fori_loop(0, _NVEC, _a_vec, 0)
        return 0

    lax.fori_loop(0, shard_chunks, _a_chunk, 0)
    pltpu.sync_copy(cnt_v, counts_sp.at[sub])
    plsc.subcore_barrier()

    pltpu.sync_copy(counts_sp, hist_v)

    def _b_step(sv, carry):
        cnts = plsc.load_gather(hist_v, [tl, jnp.zeros((16,), i32) + sv])
        cap = (cnts + 7) & ~7
        ps = plsc.cumsum(cap)
        base_v[pl.ds(sv * 16, 16)] = carry + ps - cap
        return carry + jnp.max(ps)

    lax.fori_loop(0, ngroup * 16, _b_step, jnp.zeros((16,), i32))

    def _b2_step(g, _):
        sidx = g * 16 + tl
        nx = plsc.load_gather(base_v, [sidx * 16 + sub])
        next_v[pl.ds(g * 16, 16)] = nx
        start_v[pl.ds(g * 16, 16)] = nx
        myc = plsc.load_gather(hist_v, [jnp.zeros((16,), i32) + sub, sidx])
        cnt_v[pl.ds(g * 16, 16)] = myc
        return 0

    lax.fori_loop(0, ngroup, _b2_step, 0)

    def _c_chunk(ch, _):
        off = pl.multiple_of(shard0 + ch * _CHUNK, _CHUNK)
        pltpu.sync_copy(keys_ref.at[pl.ds(off, _CHUNK)], kbuf)

        def _c_vec(i, _):
            kv = kbuf[pl.ds(i * 16, 16)]
            st = (kv & _KMASK) >> _SB
            rank, last = plsc.scan_count(st)
            cur = plsc.load_gather(next_v, [st])
            slot = cur + rank - _RB
            plsc.addupdate_scatter(next_v, [st], rank + (1 - _RB), mask=last)
            pltpu.sync_copy(kbuf.at[pl.ds(i * 16, 16)], bucket_sp.at[slot])
            return 0

        lax.fori_loop(0, _NVEC, _c_vec, 0)
        return 0

    lax.fori_loop(0, shard_chunks, _c_chunk, 0)
    plsc.subcore_barrier()

    tile_cells = _STRIP // 16

    def _d_strip(s, _):
        for z in range(tile_cells // _ZB):
            zoff = pl.multiple_of(sub * tile_cells + z * _ZB, _ZB)
            pltpu.sync_copy(zbuf, strip_sp.at[pl.ds(zoff, _ZB)])
        plsc.subcore_barrier()
        start = pl.multiple_of(_scal(start_v, s), 8)
        cnt = _scal(cnt_v, s)
        sbase = s << _SB

        def _d_chunk(ch, _):
            coff = pl.multiple_of(start + ch * _CHUNK, 8)
            pltpu.sync_copy(bucket_sp.at[pl.ds(coff, _CHUNK)], kbuf)
            vleft = cnt - ch * _CHUNK

            def _d_vec(i, _):
                kv = kbuf[pl.ds(i * 16, 16)]
                cell = (kv & _KMASK) - sbase
                valid = ((i * 16 + tl) < vleft) & (cell >= 0) & (cell < _STRIP)
                off = jnp.where(valid, cell, tl)
                is_m2 = (kv >> 27) & 1
                is_pad = (kv >> 28) & 1
                val = jnp.where(
                    valid & (is_pad == 0),
                    jnp.where(is_m2 == 1, jnp.float32(_M2UNIT),
                              jnp.float32(1.0)),
                    jnp.float32(0.0))
                valbuf[pl.ds(i * 16, 16)] = val
                pltpu.sync_copy(valbuf.at[pl.ds(i * 16, 16)],
                                strip_sp.at[off], add=True)
                return 0

            lax.fori_loop(0, _NVEC, _d_vec, 0)
            return 0

        lax.fori_loop(0, (cnt + _CHUNK - 1) // _CHUNK, _d_chunk, 0)
        plsc.subcore_barrier()
        woff = pl.multiple_of((s << _SB) + sub * tile_cells, _ZB)
        soff = pl.multiple_of(sub * tile_cells, _ZB)
        pltpu.sync_copy(strip_sp.at[pl.ds(soff, tile_cells)],
                        plane_ref.at[pl.ds(woff, tile_cells)])
        plsc.subcore_barrier()
        return 0

    lax.fori_loop(0, nstrips, _d_strip, 0)


def _sc_body(k1_ref, k2_ref, kx_ref, p1_ref, p2_ref, x_ref, strip_sp,
             bucket_sp, counts_sp, kbuf, valbuf, zbuf, hist_v, cnt_v, base_v,
             next_v, start_v):
    core = lax.axis_index("c")
    sub = lax.axis_index("s")
    refs = (strip_sp, bucket_sp, counts_sp, kbuf, valbuf, zbuf, hist_v, cnt_v,
            base_v, next_v, start_v)

    def _zb_init(i, _):
        zbuf[pl.ds(i * 16, 16)] = jnp.zeros((16,), jnp.float32)
        return 0

    lax.fori_loop(0, _ZB // 16, _zb_init, 0)

    @pl.when(core == 0)
    def _():
        _build_plane(k1_ref, p1_ref, sub, refs, shard_chunks=20,
                     nstrips=_NS_P)

    @pl.when(core == 1)
    def _():
        _build_plane(k2_ref, p2_ref, sub, refs, shard_chunks=12,
                     nstrips=_NS_P)
        _build_plane(kx_ref, x_ref, sub, refs, shard_chunks=16, nstrips=5)


def _sc_build_planes(k1, k2, kx):
    mesh = plsc.VectorSubcoreMesh(core_axis_name="c", subcore_axis_name="s")
    f32 = jnp.float32
    return pl.kernel(
        _sc_body,
        out_type=(
            jax.ShapeDtypeStruct((_PLANE,), f32),
            jax.ShapeDtypeStruct((_PLANE,), f32),
            jax.ShapeDtypeStruct((_XPAD,), f32),
        ),
        mesh=mesh,
        scratch_types=[
            pltpu.VMEM_SHARED((_STRIP,), f32),
            pltpu.VMEM_SHARED((681984,), jnp.int32),
            pltpu.VMEM_SHARED((16, 256), jnp.int32),
            pltpu.VMEM((_CHUNK,), jnp.int32),
            pltpu.VMEM((_CHUNK,), f32),
            pltpu.VMEM((_ZB,), f32),
            pltpu.VMEM((16, 256), jnp.int32),
            pltpu.VMEM((256,), jnp.int32),
            pltpu.VMEM((4096,), jnp.int32),
            pltpu.VMEM((256,), jnp.int32),
            pltpu.VMEM((256,), jnp.int32),
        ],
        compiler_params=pltpu.CompilerParams(needs_layout_passes=False),
    )(k1, k2, kx)


def _h_kernel(x_ref, w1_ref, o_ref):
    x = jnp.minimum(x_ref[...], 1.0)
    o_ref[...] = lax.dot_general(
        x, w1_ref[...], (((1,), (0,)), ((), ())),
        preferred_element_type=jnp.float32)


def _decode(p1, p2):
    hit = (p2 >= _M2UNIT) | ((p2 > 0.0) & (p1 > 0.0))
    return jnp.where(hit, jnp.float32(1.0), jnp.float32(0.0))


def _p1_kernel(p1_ref, p2_ref, h_ref, w2_ref, h2_ref, deg_ref, u8_ref,
               acc_ref, dacc_ref, *, nk):
    k = pl.program_id(1)

    @pl.when(k == 0)
    def _():
        acc_ref[...] = jnp.zeros_like(acc_ref)
        dacc_ref[...] = jnp.zeros_like(dacc_ref)

    m = _decode(p1_ref[...], p2_ref[...])
    u8_ref[...] = m.astype(jnp.uint8)
    acc_ref[...] += lax.dot_general(
        m, h_ref[...], (((1,), (0,)), ((), ())),
        preferred_element_type=jnp.float32)
    dacc_ref[...] += jnp.sum(m, axis=1, keepdims=True)

    @pl.when(k == nk - 1)
    def _():
        deg = dacc_ref[...] + 1.0
        h1 = jax.nn.relu(acc_ref[...] / deg)
        h2_ref[...] = lax.dot_general(
            h1, w2_ref[...], (((1,), (0,)), ((), ())),
            preferred_element_type=jnp.float32)
        deg_ref[...] = deg


def _p2_kernel(u8_ref, h2_ref, deg_ref, o_ref, acc_ref, *, nk):
    k = pl.program_id(1)

    @pl.when(k == 0)
    def _():
        acc_ref[...] = jnp.zeros_like(acc_ref)

    m = u8_ref[...].astype(jnp.float32)
    acc_ref[...] += lax.dot_general(
        m, h2_ref[...], (((1,), (0,)), ((), ())),
        preferred_element_type=jnp.float32)

    @pl.when(k == nk - 1)
    def _():
        o_ref[...] = acc_ref[...] / deg_ref[...]


def kernel(attr_idx, edge_idx, S_mask1_idx, S_mask2_idx, W1, W2, n, d, n0):
    ds = W1.shape[0]
    ns = 10000
    f1 = W1.shape[1]
    f2 = W2.shape[1]
    i32 = jnp.int32

    k1 = (S_mask1_idx[0] * _NP + S_mask1_idx[1]).astype(i32)
    ke = (edge_idx[0] * _NP + edge_idx[1]).astype(i32)
    km2 = ((S_mask2_idx[0] * _NP + S_mask2_idx[1]) | (1 << 27)).astype(i32)
    kx = (attr_idx[0] * 256 + (attr_idx[1] & 255)).astype(i32)
    k1 = _pad_keys(k1, 20)
    k2 = _pad_keys(jnp.concatenate([ke, km2]), 12)
    kx = _pad_keys(kx, 16)

    p1f, p2f, xf = _sc_build_planes(k1, k2, kx)
    p1 = p1f.reshape(_NP, _NP)
    p2 = p2f.reshape(_NP, _NP)
    xp = xf.reshape(_XPAD // 256, 256)

    ni = _NP // _RT
    nk = _NP // _KC

    h = pl.pallas_call(
        _h_kernel,
        grid=(ni,),
        in_specs=[
            pl.BlockSpec((_RT, ds), lambda i: (i, 0)),
            pl.BlockSpec((ds, f1), lambda i: (0, 0)),
        ],
        out_specs=pl.BlockSpec((_RT, f1), lambda i: (i, 0)),
        out_shape=jax.ShapeDtypeStruct((_NP, f1), jnp.float32),
    )(xp[:_NP], W1)

    h2, deg, u8 = pl.pallas_call(
        functools.partial(_p1_kernel, nk=nk),
        grid=(ni, nk),
        in_specs=[
            pl.BlockSpec((_RT, _KC), lambda i, k: (i, k)),
            pl.BlockSpec((_RT, _KC), lambda i, k: (i, k)),
            pl.BlockSpec((_KC, f1), lambda i, k: (k, 0)),
            pl.BlockSpec((f1, f2), lambda i, k: (0, 0)),
        ],
        out_specs=[
            pl.BlockSpec((_RT, f2), lambda i, k: (i, 0)),
            pl.BlockSpec((_RT, 1), lambda i, k: (i, 0)),
            pl.BlockSpec((_RT, _KC), lambda i, k: (i, k)),
        ],
        out_shape=[
            jax.ShapeDtypeStruct((_NP, f2), jnp.float32),
            jax.ShapeDtypeStruct((_NP, 1), jnp.float32),
            jax.ShapeDtypeStruct((_NP, _NP), jnp.uint8),
        ],
        scratch_shapes=[
            pltpu.VMEM((_RT, f1), jnp.float32),
            pltpu.VMEM((_RT, 1), jnp.float32),
        ],
    )(p1, p2, h, W2)

    out = pl.pallas_call(
        functools.partial(_p2_kernel, nk=nk),
        grid=(ni, nk),
        in_specs=[
            pl.BlockSpec((_RT, _KC), lambda i, k: (i, k)),
            pl.BlockSpec((_KC, f2), lambda i, k: (k, 0)),
            pl.BlockSpec((_RT, 1), lambda i, k: (i, 0)),
        ],
        out_specs=pl.BlockSpec((_RT, f2), lambda i, k: (i, 0)),
        out_shape=jax.ShapeDtypeStruct((_NP, f2), jnp.float32),
        scratch_shapes=[pltpu.VMEM((_RT, f2), jnp.float32)],
    )(u8, h2, deg)

    return out[:ns]


# speedup vs baseline: 2.9095x; 1.0339x over previous
import functools

import jax
import jax.numpy as jnp
from jax import lax
from jax.experimental import pallas as pl
from jax.experimental.pallas import tpu as pltpu
from jax.experimental.pallas import tpu_sc as plsc

_NP = 10240
_RT = 1024
_KC = 1024
_PLANE = _NP * _NP
_SB = 19
_STRIP = 1 << _SB
_NS_P = _PLANE >> _SB
_XPAD = 5 << _SB
_M2UNIT = float(1 << 19)
_KMASK = (1 << 27) - 1
_CHUNK = 2048
_NVEC = _CHUNK // 16
_RB = 1
_ZB = 16384


def _pad_keys(k, shard_chunks):
    total = shard_chunks * _CHUNK * 16
    pad = total - k.shape[0]
    padk = ((jnp.arange(pad, dtype=jnp.int32) & 8191) * 64) | (1 << 28)
    return jnp.concatenate([k, padk])


def _scal(ref, idx):
    base = pl.multiple_of((idx // 16) * 16, 16)
    vec = ref[pl.ds(base, 16)]
    lane = lax.iota(jnp.int32, 16) == (idx % 16)
    return jnp.max(jnp.where(lane, vec, jnp.int32(0)))


def _build_plane(keys_ref, plane_ref, sub, refs, *, shard_chunks, nstrips):
    (strip_sp, bucket_sp, counts_sp, kbuf, valbuf, zbuf, hist_v, cnt_v,
     base_v, next_v, start_v, idx2, key2, val2, sem) = refs
    tl = lax.iota(jnp.int32, 16)
    i32 = jnp.int32
    ngroup = (nstrips + 15) // 16

    def _zero16(g, _):
        cnt_v[pl.ds(g * 16, 16)] = jnp.zeros((16,), i32)
        return 0

    lax.fori_loop(0, 16, _zero16, 0)
    shard0 = sub * shard_chunks * _CHUNK

    def _a_chunk(ch, _):
        off = pl.multiple_of(shard0 + ch * _CHUNK, _CHUNK)
        pltpu.sync_copy(keys_ref.at[pl.ds(off, _CHUNK)], kbuf)

        def _a_vec(i, _):
            kv = kbuf[pl.ds(i * 16, 16)]
            st = (kv & _KMASK) >> _SB
            rank, last = plsc.scan_count(st)
            plsc.addupdate_scatter(cnt_v, [st], rank + (1 - _RB), mask=last)
            return 0

        lax.fori_loop(0, _NVEC, _a_vec, 0)
        return 0

    lax.fori_loop(0, shard_chunks, _a_chunk, 0)
    pltpu.sync_copy(cnt_v, counts_sp.at[sub])
    plsc.subcore_barrier()

    pltpu.sync_copy(counts_sp, hist_v)

    def _b_step(sv, carry):
        cnts = plsc.load_gather(hist_v, [tl, jnp.zeros((16,), i32) + sv])
        cap = (cnts + 7) & ~7
        ps = plsc.cumsum(cap)
        base_v[pl.ds(sv * 16, 16)] = carry + ps - cap
        return carry + jnp.max(ps)

    lax.fori_loop(0, ngroup * 16, _b_step, jnp.zeros((16,), i32))

    def _b2_step(g, _):
        sidx = g * 16 + tl
        nx = plsc.load_gather(base_v, [sidx * 16 + sub])
        next_v[pl.ds(g * 16, 16)] = nx
        start_v[pl.ds(g * 16, 16)] = nx
        myc = plsc.load_gather(hist_v, [jnp.zeros((16,), i32) + sub, sidx])
        cnt_v[pl.ds(g * 16, 16)] = myc
        return 0

    lax.fori_loop(0, ngroup, _b2_step, 0)

    def _c_chunk(ch, _):
        off = pl.multiple_of(shard0 + ch * _CHUNK, _CHUNK)
        pltpu.sync_copy(keys_ref.at[pl.ds(off, _CHUNK)], kbuf)

        def _c_vec(i, _):
            kv = kbuf[pl.ds(i * 16, 16)]
            st = (kv & _KMASK) >> _SB
            rank, last = plsc.scan_count(st)
            cur = plsc.load_gather(next_v, [st])
            slot = cur + rank - _RB
            plsc.addupdate_scatter(next_v, [st], rank + (1 - _RB), mask=last)
            idx2[i // 8, pl.ds((i % 8) * 16, 16)] = slot
            key2[i // 8, pl.ds((i % 8) * 16, 16)] = kv
            return 0

        lax.fori_loop(0, _NVEC, _c_vec, 0)
        for j in range(16):
            pltpu.async_copy(key2.at[j], bucket_sp.at[idx2.at[j]], sem)

        def _c_drain(j, _):
            pltpu.make_async_copy(key2.at[0], bucket_sp.at[idx2.at[0]],
                                  sem).wait()
            return 0

        lax.fori_loop(0, 16, _c_drain, 0)
        return 0

    lax.fori_loop(0, shard_chunks, _c_chunk, 0)
    plsc.subcore_barrier()

    tile_cells = _STRIP // 16

    def _z(z, _):
        zoff = pl.multiple_of(sub * tile_cells + z * _ZB, _ZB)
        pltpu.sync_copy(zbuf, strip_sp.at[pl.ds(zoff, _ZB)])
        return 0

    lax.fori_loop(0, tile_cells // _ZB, _z, 0)
    plsc.subcore_barrier()

    def _d_strip(s, _):
        start = pl.multiple_of(_scal(start_v, s), 8)
        cnt = _scal(cnt_v, s)
        sbase = s << _SB

        def _make_chunk(adding):
            def _d_chunk(ch, _):
                coff = pl.multiple_of(start + ch * _CHUNK, 8)
                pltpu.sync_copy(bucket_sp.at[pl.ds(coff, _CHUNK)], kbuf)
                vleft = cnt - ch * _CHUNK

                def _d_vec(i, _):
                    kv = kbuf[pl.ds(i * 16, 16)]
                    cell = (kv & _KMASK) - sbase
                    valid = (((i * 16 + tl) < vleft) & (cell >= 0)
                             & (cell < _STRIP))
                    off = jnp.where(valid, cell, tl)
                    idx2[i // 8, pl.ds((i % 8) * 16, 16)] = off
                    if adding:
                        is_m2 = (kv >> 27) & 1
                        is_pad = (kv >> 28) & 1
                        val = jnp.where(
                            valid & (is_pad == 0),
                            jnp.where(is_m2 == 1, jnp.float32(_M2UNIT),
                                      jnp.float32(1.0)),
                            jnp.float32(0.0))
                    else:
                        val = jnp.zeros((16,), jnp.float32)
                    val2[i // 8, pl.ds((i % 8) * 16, 16)] = val
                    return 0

                lax.fori_loop(0, _NVEC, _d_vec, 0)
                for j in range(16):
                    pltpu.async_copy(val2.at[j], strip_sp.at[idx2.at[j]],
                                     sem, add=adding)

                def _d_drain(j, _):
                    pltpu.make_async_copy(val2.at[0],
                                          strip_sp.at[idx2.at[0]],
                                          sem).wait()
                    return 0

                lax.fori_loop(0, 16, _d_drain, 0)
                return 0

            return _d_chunk

        nch = (cnt + _CHUNK - 1) // _CHUNK
        lax.fori_loop(0, nch, _make_chunk(True), 0)
        plsc.subcore_barrier()
        woff = pl.multiple_of((s << _SB) + sub * tile_cells, _ZB)
        soff = pl.multiple_of(sub * tile_cells, _ZB)
        pltpu.sync_copy(strip_sp.at[pl.ds(soff, tile_cells)],
                        plane_ref.at[pl.ds(woff, tile_cells)])
        plsc.subcore_barrier()
        lax.fori_loop(0, nch, _make_chunk(False), 0)
        plsc.subcore_barrier()
        return 0

    lax.fori_loop(0, nstrips, _d_strip, 0)


def _sc_body(k1_ref, k2_ref, kx_ref, p1_ref, p2_ref, x_ref, strip_sp,
             bucket_sp, counts_sp, kbuf, valbuf, zbuf, hist_v, cnt_v, base_v,
             next_v, start_v, idx2, key2, val2, sem):
    core = lax.axis_index("c")
    sub = lax.axis_index("s")
    refs = (strip_sp, bucket_sp, counts_sp, kbuf, valbuf, zbuf, hist_v, cnt_v,
            base_v, next_v, start_v, idx2, key2, val2, sem)

    def _zb_init(i, _):
        zbuf[pl.ds(i * 16, 16)] = jnp.zeros((16,), jnp.float32)
        return 0

    lax.fori_loop(0, _ZB // 16, _zb_init, 0)

    @pl.when(core == 0)
    def _():
        _build_plane(k1_ref, p1_ref, sub, refs, shard_chunks=20,
                     nstrips=_NS_P)

    @pl.when(core == 1)
    def _():
        _build_plane(k2_ref, p2_ref, sub, refs, shard_chunks=12,
                     nstrips=_NS_P)
        _build_plane(kx_ref, x_ref, sub, refs, shard_chunks=16, nstrips=5)


def _sc_build_planes(k1, k2, kx):
    mesh = plsc.VectorSubcoreMesh(core_axis_name="c", subcore_axis_name="s")
    f32 = jnp.float32
    return pl.kernel(
        _sc_body,
        out_type=(
            jax.ShapeDtypeStruct((_PLANE,), f32),
            jax.ShapeDtypeStruct((_PLANE,), f32),
            jax.ShapeDtypeStruct((_XPAD,), f32),
        ),
        mesh=mesh,
        scratch_types=[
            pltpu.VMEM_SHARED((_STRIP,), f32),
            pltpu.VMEM_SHARED((681984,), jnp.int32),
            pltpu.VMEM_SHARED((16, 256), jnp.int32),
            pltpu.VMEM((_CHUNK,), jnp.int32),
            pltpu.VMEM((_CHUNK,), f32),
            pltpu.VMEM((_ZB,), f32),
            pltpu.VMEM((16, 256), jnp.int32),
            pltpu.VMEM((256,), jnp.int32),
            pltpu.VMEM((4096,), jnp.int32),
            pltpu.VMEM((256,), jnp.int32),
            pltpu.VMEM((256,), jnp.int32),
            pltpu.VMEM((16, 128), jnp.int32),
            pltpu.VMEM((16, 128), jnp.int32),
            pltpu.VMEM((16, 128), jnp.float32),
            pltpu.SemaphoreType.DMA,
        ],
        compiler_params=pltpu.CompilerParams(needs_layout_passes=False),
    )(k1, k2, kx)


def _h_kernel(x_ref, w1_ref, o_ref):
    x = jnp.minimum(x_ref[...], 1.0)
    o_ref[...] = lax.dot_general(
        x, w1_ref[...], (((1,), (0,)), ((), ())),
        preferred_element_type=jnp.float32)


def _decode(p1, p2):
    hit = (p2 >= _M2UNIT) | ((p2 > 0.0) & (p1 > 0.0))
    return jnp.where(hit, jnp.float32(1.0), jnp.float32(0.0))


def _p1_kernel(p1_ref, p2_ref, h_ref, w2_ref, h2_ref, deg_ref, u8_ref,
               acc_ref, dacc_ref, *, nk):
    k = pl.program_id(1)

    @pl.when(k == 0)
    def _():
        acc_ref[...] = jnp.zeros_like(acc_ref)
        dacc_ref[...] = jnp.zeros_like(dacc_ref)

    m = _decode(p1_ref[...], p2_ref[...])
    u8_ref[...] = m.astype(jnp.uint8)
    acc_ref[...] += lax.dot_general(
        m, h_ref[...], (((1,), (0,)), ((), ())),
        preferred_element_type=jnp.float32)
    dacc_ref[...] += jnp.sum(m, axis=1, keepdims=True)

    @pl.when(k == nk - 1)
    def _():
        deg = dacc_ref[...] + 1.0
        h1 = jax.nn.relu(acc_ref[...] / deg)
        h2_ref[...] = lax.dot_general(
            h1, w2_ref[...], (((1,), (0,)), ((), ())),
            preferred_element_type=jnp.float32)
        deg_ref[...] = deg


def _p2_kernel(u8_ref, h2_ref, deg_ref, o_ref, acc_ref, *, nk):
    k = pl.program_id(1)

    @pl.when(k == 0)
    def _():
        acc_ref[...] = jnp.zeros_like(acc_ref)

    m = u8_ref[...].astype(jnp.float32)
    acc_ref[...] += lax.dot_general(
        m, h2_ref[...], (((1,), (0,)), ((), ())),
        preferred_element_type=jnp.float32)

    @pl.when(k == nk - 1)
    def _():
        o_ref[...] = acc_ref[...] / deg_ref[...]


def kernel(attr_idx, edge_idx, S_mask1_idx, S_mask2_idx, W1, W2, n, d, n0):
    ds = W1.shape[0]
    ns = 10000
    f1 = W1.shape[1]
    f2 = W2.shape[1]
    i32 = jnp.int32

    k1 = (S_mask1_idx[0] * _NP + S_mask1_idx[1]).astype(i32)
    ke = (edge_idx[0] * _NP + edge_idx[1]).astype(i32)
    km2 = ((S_mask2_idx[0] * _NP + S_mask2_idx[1]) | (1 << 27)).astype(i32)
    kx = (attr_idx[0] * 256 + (attr_idx[1] & 255)).astype(i32)
    k1 = _pad_keys(k1, 20)
    k2 = _pad_keys(jnp.concatenate([ke, km2]), 12)
    kx = _pad_keys(kx, 16)

    p1f, p2f, xf = _sc_build_planes(k1, k2, kx)
    p1 = p1f.reshape(_NP, _NP)
    p2 = p2f.reshape(_NP, _NP)
    xp = xf.reshape(_XPAD // 256, 256)

    ni = _NP // _RT
    nk = _NP // _KC

    h = pl.pallas_call(
        _h_kernel,
        grid=(ni,),
        in_specs=[
            pl.BlockSpec((_RT, ds), lambda i: (i, 0)),
            pl.BlockSpec((ds, f1), lambda i: (0, 0)),
        ],
        out_specs=pl.BlockSpec((_RT, f1), lambda i: (i, 0)),
        out_shape=jax.ShapeDtypeStruct((_NP, f1), jnp.float32),
    )(xp[:_NP], W1)

    h2, deg, u8 = pl.pallas_call(
        functools.partial(_p1_kernel, nk=nk),
        grid=(ni, nk),
        in_specs=[
            pl.BlockSpec((_RT, _KC), lambda i, k: (i, k)),
            pl.BlockSpec((_RT, _KC), lambda i, k: (i, k)),
            pl.BlockSpec((_KC, f1), lambda i, k: (k, 0)),
            pl.BlockSpec((f1, f2), lambda i, k: (0, 0)),
        ],
        out_specs=[
            pl.BlockSpec((_RT, f2), lambda i, k: (i, 0)),
            pl.BlockSpec((_RT, 1), lambda i, k: (i, 0)),
            pl.BlockSpec((_RT, _KC), lambda i, k: (i, k)),
        ],
        out_shape=[
            jax.ShapeDtypeStruct((_NP, f2), jnp.float32),
            jax.ShapeDtypeStruct((_NP, 1), jnp.float32),
            jax.ShapeDtypeStruct((_NP, _NP), jnp.uint8),
        ],
        scratch_shapes=[
            pltpu.VMEM((_RT, f1), jnp.float32),
            pltpu.VMEM((_RT, 1), jnp.float32),
        ],
    )(p1, p2, h, W2)

    out = pl.pallas_call(
        functools.partial(_p2_kernel, nk=nk),
        grid=(ni, nk),
        in_specs=[
            pl.BlockSpec((_RT, _KC), lambda i, k: (i, k)),
            pl.BlockSpec((_KC, f2), lambda i, k: (k, 0)),
            pl.BlockSpec((_RT, 1), lambda i, k: (i, 0)),
        ],
        out_specs=pl.BlockSpec((_RT, f2), lambda i, k: (i, 0)),
        out_shape=jax.ShapeDtypeStruct((_NP, f2), jnp.float32),
        scratch_shapes=[pltpu.VMEM((_RT, f2), jnp.float32)],
    )(u8, h2, deg)

    return out[:ns]


# speedup vs baseline: 5.2907x; 1.8184x over previous
import functools

import jax
import jax.numpy as jnp
from jax import lax
from jax.experimental import pallas as pl
from jax.experimental.pallas import tpu as pltpu
from jax.experimental.pallas import tpu_sc as plsc

_NP = 10240
_RT = 1024
_KC = 1024
_PLANE = _NP * _NP
_SB = 19
_STRIP = 1 << _SB
_NS_P = _PLANE >> _SB
_XPAD = 5 << _SB
_M2UNIT = float(1 << 19)
_KMASK = (1 << 27) - 1
_CHUNK = 2048
_NVEC = _CHUNK // 16
_RB = 1
_ZB = 16384


def _pad_keys(k, shard_chunks):
    total = shard_chunks * _CHUNK * 16
    pad = total - k.shape[0]
    padk = ((jnp.arange(pad, dtype=jnp.int32) & 8191) * 64) | (1 << 28)
    return jnp.concatenate([k, padk])


def _scal(ref, idx):
    base = pl.multiple_of((idx // 16) * 16, 16)
    vec = ref[pl.ds(base, 16)]
    lane = lax.iota(jnp.int32, 16) == (idx % 16)
    return jnp.max(jnp.where(lane, vec, jnp.int32(0)))


def _build_plane(keys_ref, plane_ref, sub, refs, *, shard_chunks, nstrips):
    (strip_sp, bucket_sp, counts_sp, kbuf, valbuf, zbuf, hist_v, cnt_v,
     base_v, next_v, start_v, idx2, key2, val2, sem) = refs
    tl = lax.iota(jnp.int32, 16)
    i32 = jnp.int32
    ngroup = (nstrips + 15) // 16

    def _zero16(g, _):
        cnt_v[pl.ds(g * 16, 16)] = jnp.zeros((16,), i32)
        return 0

    lax.fori_loop(0, 16, _zero16, 0)
    shard0 = sub * shard_chunks * _CHUNK

    def _a_chunk(ch, _):
        off = pl.multiple_of(shard0 + ch * _CHUNK, _CHUNK)
        pltpu.sync_copy(keys_ref.at[pl.ds(off, _CHUNK)], kbuf)

        def _a_vec(i, _):
            kv = kbuf[pl.ds(i * 16, 16)]
            st = (kv & _KMASK) >> _SB
            rank, last = plsc.scan_count(st)
            plsc.addupdate_scatter(cnt_v, [st], rank + (1 - _RB), mask=last)
            return 0

        lax.fori_loop(0, _NVEC, _a_vec, 0)
        return 0

    lax.fori_loop(0, shard_chunks, _a_chunk, 0)
    pltpu.sync_copy(cnt_v, counts_sp.at[sub])
    plsc.subcore_barrier()

    pltpu.sync_copy(counts_sp, hist_v)

    def _b_step(sv, carry):
        cnts = plsc.load_gather(hist_v, [tl, jnp.zeros((16,), i32) + sv])
        cap = (cnts + 7) & ~7
        ps = plsc.cumsum(cap)
        base_v[pl.ds(sv * 16, 16)] = carry + ps - cap
        return carry + jnp.max(ps)

    lax.fori_loop(0, ngroup * 16, _b_step, jnp.zeros((16,), i32))

    def _b2_step(g, _):
        sidx = g * 16 + tl
        nx = plsc.load_gather(base_v, [sidx * 16 + sub])
        next_v[pl.ds(g * 16, 16)] = nx
        start_v[pl.ds(g * 16, 16)] = nx
        myc = plsc.load_gather(hist_v, [jnp.zeros((16,), i32) + sub, sidx])
        cnt_v[pl.ds(g * 16, 16)] = myc
        return 0

    lax.fori_loop(0, ngroup, _b2_step, 0)

    def _c_chunk(ch, _):
        off = pl.multiple_of(shard0 + ch * _CHUNK, _CHUNK)
        pltpu.sync_copy(keys_ref.at[pl.ds(off, _CHUNK)], kbuf)

        def _c_vec(i, _):
            kv = kbuf[pl.ds(i * 16, 16)]
            st = (kv & _KMASK) >> _SB
            rank, last = plsc.scan_count(st)
            cur = plsc.load_gather(next_v, [st])
            slot = cur + rank - _RB
            plsc.addupdate_scatter(next_v, [st], rank + (1 - _RB), mask=last)
            idx2[i // 8, pl.ds((i % 8) * 16, 16)] = slot
            key2[i // 8, pl.ds((i % 8) * 16, 16)] = kv
            return 0

        lax.fori_loop(0, _NVEC, _c_vec, 0)
        for j in range(16):
            pltpu.async_copy(key2.at[j], bucket_sp.at[idx2.at[j]], sem)

        def _c_drain(j, _):
            pltpu.make_async_copy(key2.at[0], bucket_sp.at[idx2.at[0]],
                                  sem).wait()
            return 0

        lax.fori_loop(0, 16, _c_drain, 0)
        return 0

    lax.fori_loop(0, shard_chunks, _c_chunk, 0)
    plsc.subcore_barrier()

    tile_cells = _STRIP // 16

    def _z(z, _):
        zoff = pl.multiple_of(sub * tile_cells + z * _ZB, _ZB)
        pltpu.sync_copy(zbuf, strip_sp.at[pl.ds(zoff, _ZB)])
        return 0

    lax.fori_loop(0, tile_cells // _ZB, _z, 0)
    plsc.subcore_barrier()

    def _d_strip(s, _):
        start = pl.multiple_of(_scal(start_v, s), 8)
        cnt = _scal(cnt_v, s)
        sbase = s << _SB

        def _make_chunk(adding):
            def _d_chunk(ch, _):
                coff = pl.multiple_of(start + ch * _CHUNK, 8)
                pltpu.sync_copy(bucket_sp.at[pl.ds(coff, _CHUNK)], kbuf)
                vleft = cnt - ch * _CHUNK

                def _d_vec(i, _):
                    kv = kbuf[pl.ds(i * 16, 16)]
                    cell = (kv & _KMASK) - sbase
                    valid = (((i * 16 + tl) < vleft) & (cell >= 0)
                             & (cell < _STRIP))
                    off = jnp.where(valid, cell, tl)
                    idx2[i // 8, pl.ds((i % 8) * 16, 16)] = off
                    if adding:
                        is_m2 = (kv >> 27) & 1
                        is_pad = (kv >> 28) & 1
                        val = jnp.where(
                            valid & (is_pad == 0),
                            jnp.where(is_m2 == 1, jnp.float32(_M2UNIT),
                                      jnp.float32(1.0)),
                            jnp.float32(0.0))
                    else:
                        val = jnp.zeros((16,), jnp.float32)
                    val2[i // 8, pl.ds((i % 8) * 16, 16)] = val
                    return 0

                used = jnp.clip(vleft, 0, _CHUNK)
                nrow = (used + 127) // 128
                lax.fori_loop(0, nrow * 8, _d_vec, 0)

                def _d_fire(j, _):
                    pltpu.async_copy(val2.at[j], strip_sp.at[idx2.at[j]],
                                     sem, add=adding)
                    return 0

                lax.fori_loop(0, nrow, _d_fire, 0)

                def _d_drain(j, _):
                    pltpu.make_async_copy(val2.at[0],
                                          strip_sp.at[idx2.at[0]],
                                          sem).wait()
                    return 0

                lax.fori_loop(0, nrow, _d_drain, 0)
                return 0

            return _d_chunk

        nch = (cnt + _CHUNK - 1) // _CHUNK
        lax.fori_loop(0, nch, _make_chunk(True), 0)
        plsc.subcore_barrier()
        woff = pl.multiple_of((s << _SB) + sub * tile_cells, _ZB)
        soff = pl.multiple_of(sub * tile_cells, _ZB)
        pltpu.sync_copy(strip_sp.at[pl.ds(soff, tile_cells)],
                        plane_ref.at[pl.ds(woff, tile_cells)])
        plsc.subcore_barrier()
        lax.fori_loop(0, nch, _make_chunk(False), 0)
        plsc.subcore_barrier()
        return 0

    lax.fori_loop(0, nstrips, _d_strip, 0)


def _sc_body(k1_ref, k2_ref, kx_ref, p1_ref, p2_ref, x_ref, strip_sp,
             bucket_sp, counts_sp, kbuf, valbuf, zbuf, hist_v, cnt_v, base_v,
             next_v, start_v, idx2, key2, val2, sem):
    core = lax.axis_index("c")
    sub = lax.axis_index("s")
    refs = (strip_sp, bucket_sp, counts_sp, kbuf, valbuf, zbuf, hist_v, cnt_v,
            base_v, next_v, start_v, idx2, key2, val2, sem)

    def _zb_init(i, _):
        zbuf[pl.ds(i * 16, 16)] = jnp.zeros((16,), jnp.float32)
        return 0

    lax.fori_loop(0, _ZB // 16, _zb_init, 0)

    @pl.when(core == 0)
    def _():
        _build_plane(k1_ref, p1_ref, sub, refs, shard_chunks=20,
                     nstrips=_NS_P)

    @pl.when(core == 1)
    def _():
        _build_plane(k2_ref, p2_ref, sub, refs, shard_chunks=12,
                     nstrips=_NS_P)
        _build_plane(kx_ref, x_ref, sub, refs, shard_chunks=16, nstrips=5)


def _sc_build_planes(k1, k2, kx):
    mesh = plsc.VectorSubcoreMesh(core_axis_name="c", subcore_axis_name="s")
    f32 = jnp.float32
    return pl.kernel(
        _sc_body,
        out_type=(
            jax.ShapeDtypeStruct((_PLANE,), f32),
            jax.ShapeDtypeStruct((_PLANE,), f32),
            jax.ShapeDtypeStruct((_XPAD,), f32),
        ),
        mesh=mesh,
        scratch_types=[
            pltpu.VMEM_SHARED((_STRIP,), f32),
            pltpu.VMEM_SHARED((681984,), jnp.int32),
            pltpu.VMEM_SHARED((16, 256), jnp.int32),
            pltpu.VMEM((_CHUNK,), jnp.int32),
            pltpu.VMEM((_CHUNK,), f32),
            pltpu.VMEM((_ZB,), f32),
            pltpu.VMEM((16, 256), jnp.int32),
            pltpu.VMEM((256,), jnp.int32),
            pltpu.VMEM((4096,), jnp.int32),
            pltpu.VMEM((256,), jnp.int32),
            pltpu.VMEM((256,), jnp.int32),
            pltpu.VMEM((16, 128), jnp.int32),
            pltpu.VMEM((16, 128), jnp.int32),
            pltpu.VMEM((16, 128), jnp.float32),
            pltpu.SemaphoreType.DMA,
        ],
        compiler_params=pltpu.CompilerParams(needs_layout_passes=False),
    )(k1, k2, kx)


def _h_kernel(x_ref, w1_ref, o_ref):
    x = jnp.minimum(x_ref[...], 1.0)
    o_ref[...] = lax.dot_general(
        x, w1_ref[...], (((1,), (0,)), ((), ())),
        preferred_element_type=jnp.float32)


def _decode(p1, p2):
    hit = (p2 >= _M2UNIT) | ((p2 > 0.0) & (p1 > 0.0))
    return jnp.where(hit, jnp.float32(1.0), jnp.float32(0.0))


def _p1_kernel(p1_ref, p2_ref, h_ref, w2_ref, h2_ref, deg_ref, u8_ref,
               acc_ref, dacc_ref, *, nk):
    k = pl.program_id(1)

    @pl.when(k == 0)
    def _():
        acc_ref[...] = jnp.zeros_like(acc_ref)
        dacc_ref[...] = jnp.zeros_like(dacc_ref)

    m = _decode(p1_ref[...], p2_ref[...])
    u8_ref[...] = m.astype(jnp.uint8)
    acc_ref[...] += lax.dot_general(
        m, h_ref[...], (((1,), (0,)), ((), ())),
        preferred_element_type=jnp.float32)
    dacc_ref[...] += jnp.sum(m, axis=1, keepdims=True)

    @pl.when(k == nk - 1)
    def _():
        deg = dacc_ref[...] + 1.0
        h1 = jax.nn.relu(acc_ref[...] / deg)
        h2_ref[...] = lax.dot_general(
            h1, w2_ref[...], (((1,), (0,)), ((), ())),
            preferred_element_type=jnp.float32)
        deg_ref[...] = deg


def _p2_kernel(u8_ref, h2_ref, deg_ref, o_ref, acc_ref, *, nk):
    k = pl.program_id(1)

    @pl.when(k == 0)
    def _():
        acc_ref[...] = jnp.zeros_like(acc_ref)

    m = u8_ref[...].astype(jnp.float32)
    acc_ref[...] += lax.dot_general(
        m, h2_ref[...], (((1,), (0,)), ((), ())),
        preferred_element_type=jnp.float32)

    @pl.when(k == nk - 1)
    def _():
        o_ref[...] = acc_ref[...] / deg_ref[...]


def kernel(attr_idx, edge_idx, S_mask1_idx, S_mask2_idx, W1, W2, n, d, n0):
    ds = W1.shape[0]
    ns = 10000
    f1 = W1.shape[1]
    f2 = W2.shape[1]
    i32 = jnp.int32

    k1 = (S_mask1_idx[0] * _NP + S_mask1_idx[1]).astype(i32)
    ke = (edge_idx[0] * _NP + edge_idx[1]).astype(i32)
    km2 = ((S_mask2_idx[0] * _NP + S_mask2_idx[1]) | (1 << 27)).astype(i32)
    kx = (attr_idx[0] * 256 + (attr_idx[1] & 255)).astype(i32)
    k1 = _pad_keys(k1, 20)
    k2 = _pad_keys(jnp.concatenate([ke, km2]), 12)
    kx = _pad_keys(kx, 16)

    p1f, p2f, xf = _sc_build_planes(k1, k2, kx)
    p1 = p1f.reshape(_NP, _NP)
    p2 = p2f.reshape(_NP, _NP)
    xp = xf.reshape(_XPAD // 256, 256)

    ni = _NP // _RT
    nk = _NP // _KC

    h = pl.pallas_call(
        _h_kernel,
        grid=(ni,),
        in_specs=[
            pl.BlockSpec((_RT, ds), lambda i: (i, 0)),
            pl.BlockSpec((ds, f1), lambda i: (0, 0)),
        ],
        out_specs=pl.BlockSpec((_RT, f1), lambda i: (i, 0)),
        out_shape=jax.ShapeDtypeStruct((_NP, f1), jnp.float32),
    )(xp[:_NP], W1)

    h2, deg, u8 = pl.pallas_call(
        functools.partial(_p1_kernel, nk=nk),
        grid=(ni, nk),
        in_specs=[
            pl.BlockSpec((_RT, _KC), lambda i, k: (i, k)),
            pl.BlockSpec((_RT, _KC), lambda i, k: (i, k)),
            pl.BlockSpec((_KC, f1), lambda i, k: (k, 0)),
            pl.BlockSpec((f1, f2), lambda i, k: (0, 0)),
        ],
        out_specs=[
            pl.BlockSpec((_RT, f2), lambda i, k: (i, 0)),
            pl.BlockSpec((_RT, 1), lambda i, k: (i, 0)),
            pl.BlockSpec((_RT, _KC), lambda i, k: (i, k)),
        ],
        out_shape=[
            jax.ShapeDtypeStruct((_NP, f2), jnp.float32),
            jax.ShapeDtypeStruct((_NP, 1), jnp.float32),
            jax.ShapeDtypeStruct((_NP, _NP), jnp.uint8),
        ],
        scratch_shapes=[
            pltpu.VMEM((_RT, f1), jnp.float32),
            pltpu.VMEM((_RT, 1), jnp.float32),
        ],
    )(p1, p2, h, W2)

    out = pl.pallas_call(
        functools.partial(_p2_kernel, nk=nk),
        grid=(ni, nk),
        in_specs=[
            pl.BlockSpec((_RT, _KC), lambda i, k: (i, k)),
            pl.BlockSpec((_KC, f2), lambda i, k: (k, 0)),
            pl.BlockSpec((_RT, 1), lambda i, k: (i, 0)),
        ],
        out_specs=pl.BlockSpec((_RT, f2), lambda i, k: (i, 0)),
        out_shape=jax.ShapeDtypeStruct((_NP, f2), jnp.float32),
        scratch_shapes=[pltpu.VMEM((_RT, f2), jnp.float32)],
    )(u8, h2, deg)

    return out[:ns]
